# baseline (speedup 1.0000x reference)
import functools

import jax
import jax.numpy as jnp
from jax import lax
from jax.experimental import pallas as pl
from jax.experimental.pallas import tpu as pltpu
from jax.experimental.pallas import tpu_sc as plsc

N = 10000
E = 320000
H = 128
NIN = 16
SCALE = 30.0
EPS = 1e-5

NUM_SC = 2
NUM_TILES = 16
NCHUNK = 2
ECHUNKS = (217600, 102400)
EOFFS = (0, 217600)
CHUNK = 120
NRING = 3
TAIL = 80
N_PAD = 10240
ROWS_PER_TILE = N_PAD // NUM_TILES


_EBLK = 6400

_SQRT_HALF = 0.7071067811865476


def _gelu(x):
    return 0.5 * x * (1.0 + lax.erf(x * _SQRT_HALF))


def _msg_body(het_ref, w_ref, b_ref, out_ref):
    m = lax.dot_general(het_ref[...], w_ref[...],
                        ((( 0,), (0,)), ((), ())),
                        preferred_element_type=jnp.float32)
    out_ref[...] = _gelu(m + b_ref[...])


def _edge_messages(h_E_T, W_msg, b_msg, chunk):
    echunk = ECHUNKS[chunk]
    blk0 = EOFFS[chunk] // _EBLK
    return pl.pallas_call(
        _msg_body,
        grid=(echunk // _EBLK,),
        in_specs=[
            pl.BlockSpec((H + NIN, _EBLK), lambda i: (0, i + blk0)),
            pl.BlockSpec((H + NIN, H), lambda i: (0, 0)),
            pl.BlockSpec((1, H), lambda i: (0, 0)),
        ],
        out_specs=pl.BlockSpec((_EBLK, H), lambda i: (i, 0)),
        out_shape=jax.ShapeDtypeStruct((echunk, H), jnp.float32),
    )(h_E_T, W_msg, b_msg.reshape(1, H))


def _scatter_body(msg_hbm, idx_hbm, out_hbm, rows_v, idx_v, tidx_v,
                  acc_sh, lr, li, *, eoff, ept, nfull):
    cid = lax.axis_index("c")
    sid = lax.axis_index("s")

    def _zrow(i, carry):
        for j in range(H // 16):
            rows_v[0, i, pl.ds(j * 16, 16)] = jnp.zeros((16,), jnp.float32)
        return carry

    lax.fori_loop(0, CHUNK, _zrow, 0)
    zbase = sid * ROWS_PER_TILE

    def _zcopy(k, carry):
        pltpu.sync_copy(rows_v.at[0], acc_sh.at[pl.ds(zbase + k * CHUNK, CHUNK)])
        return carry

    nz = ROWS_PER_TILE // CHUNK
    lax.fori_loop(0, nz, _zcopy, 0)
    rem = ROWS_PER_TILE - nz * CHUNK
    if rem:
        pltpu.sync_copy(rows_v.at[0, pl.ds(0, rem)],
                        acc_sh.at[pl.ds(zbase + nz * CHUNK, rem)])
    plsc.subcore_barrier()

    base = cid * (NUM_TILES * ept) + sid * ept

    def _start_load(c, b):
        off = base + c * CHUNK
        pltpu.async_copy(idx_hbm.at[pl.ds(eoff + off, CHUNK)], idx_v.at[b], li[b])
        pltpu.async_copy(msg_hbm.at[pl.ds(off, CHUNK)], rows_v.at[b], lr[b])

    def _wait_load(c, b):
        off = base + c * CHUNK
        pltpu.make_async_copy(idx_hbm.at[pl.ds(eoff + off, CHUNK)], idx_v.at[b], li[b]).wait()
        pltpu.make_async_copy(msg_hbm.at[pl.ds(off, CHUNK)], rows_v.at[b], lr[b]).wait()

    def _scatter(b):
        pltpu.sync_copy(rows_v.at[b], acc_sh.at[idx_v.at[b]], add=True)

    _start_load(0, 0)
    _start_load(1, 1)

    def _steady(c, carry, b):
        _wait_load(c, b)
        _start_load(c + 2, (b + 2) % 3)
        _scatter(b)
        return carry

    def _ring(k, carry):
        c0 = 3 * k
        for d in range(3):
            carry = _steady(c0 + d, carry, d)
        return carry

    nring = (nfull - 2) // 3
    lax.fori_loop(0, nring, _ring, 0)
    for c in range(3 * nring, nfull - 2):
        _steady(c, 0, c % 3)
    for c in (nfull - 2, nfull - 1):
        b = c % 3
        _wait_load(c, b)
        _scatter(b)
    if TAIL:
        off = base + nfull * CHUNK
        pltpu.sync_copy(idx_hbm.at[pl.ds(eoff + off, TAIL)], tidx_v)
        pltpu.sync_copy(msg_hbm.at[pl.ds(off, TAIL)], rows_v.at[2, pl.ds(0, TAIL)])
        pltpu.sync_copy(rows_v.at[2, pl.ds(0, TAIL)], acc_sh.at[tidx_v], add=True)

    plsc.subcore_barrier()
    pltpu.sync_copy(acc_sh.at[pl.ds(sid * ROWS_PER_TILE, ROWS_PER_TILE)],
                    out_hbm.at[cid, pl.ds(sid * ROWS_PER_TILE, ROWS_PER_TILE)])


def _segment_sum(h_msg, src_idx, chunk):
    ept = ECHUNKS[chunk] // (NUM_SC * NUM_TILES)
    nfull = ept // CHUNK
    assert ept - nfull * CHUNK == TAIL
    mesh = plsc.VectorSubcoreMesh(core_axis_name="c", subcore_axis_name="s")
    kern = functools.partial(
        pl.kernel,
        out_type=jax.ShapeDtypeStruct((NUM_SC, N_PAD, H), jnp.float32),
        mesh=mesh,
        scratch_types=[
            pltpu.VMEM((NRING, CHUNK, H), jnp.float32),
            pltpu.VMEM((NRING, CHUNK), jnp.int32),
            pltpu.VMEM((TAIL,), jnp.int32),
            pltpu.VMEM_SHARED((N_PAD, H), jnp.float32),
            [pltpu.SemaphoreType.DMA] * NRING,
            [pltpu.SemaphoreType.DMA] * NRING,
        ],
    )(functools.partial(_scatter_body, eoff=EOFFS[chunk], ept=ept, nfull=nfull))
    return kern(h_msg, src_idx)


_NBLK = 2000


def _ln(x, g, b):
    mu = jnp.mean(x, axis=-1, keepdims=True)
    var = jnp.mean((x - mu) ** 2, axis=-1, keepdims=True)
    return (x - mu) * lax.rsqrt(var + EPS) * g + b


def _node_body(hv_ref, *refs):
    p_refs = refs[:NCHUNK]
    (wd_ref, bd_ref, wo_ref, bo_ref,
     g1_ref, be1_ref, g2_ref, be2_ref, out_ref) = refs[NCHUNK:]
    dh = p_refs[0][0] + p_refs[0][1]
    for p in p_refs[1:]:
        dh = dh + (p[0] + p[1])
    dh = dh * (1.0 / SCALE)
    h1 = _ln(hv_ref[...] + dh, g1_ref[...], be1_ref[...])
    d = jnp.dot(h1, wd_ref[...], preferred_element_type=jnp.float32) + bd_ref[...]
    d = _gelu(d)
    d = jnp.dot(d, wo_ref[...], preferred_element_type=jnp.float32) + bo_ref[...]
    out_ref[...] = _ln(h1 + d, g2_ref[...], be2_ref[...])


def _node_update(h_V, partials, W_d, b_d, W_out, b_out, g1, be1, g2, be2):
    row = lambda v: v.reshape(1, H)
    return pl.pallas_call(
        _node_body,
        grid=(N // _NBLK,),
        in_specs=[
            pl.BlockSpec((_NBLK, H), lambda i: (i, 0)),
        ] + [
            pl.BlockSpec((NUM_SC, _NBLK, H), lambda i: (0, i, 0))
            for _ in range(NCHUNK)
        ] + [
            pl.BlockSpec((H, H), lambda i: (0, 0)),
            pl.BlockSpec((1, H), lambda i: (0, 0)),
            pl.BlockSpec((H, H), lambda i: (0, 0)),
            pl.BlockSpec((1, H), lambda i: (0, 0)),
            pl.BlockSpec((1, H), lambda i: (0, 0)),
            pl.BlockSpec((1, H), lambda i: (0, 0)),
            pl.BlockSpec((1, H), lambda i: (0, 0)),
            pl.BlockSpec((1, H), lambda i: (0, 0)),
        ],
        out_specs=pl.BlockSpec((_NBLK, H), lambda i: (i, 0)),
        out_shape=jax.ShapeDtypeStruct((N, H), jnp.float32),
    )(h_V, *partials, W_d, row(b_d), W_out, row(b_out),
      row(g1), row(be1), row(g2), row(be2))


def kernel(h_V, h_E, edge_idx, W_msg, b_msg, W_d, b_d, W_out, b_out,
           g1, be1, g2, be2):
    src_idx = edge_idx[0].astype(jnp.int32)
    h_E_T = h_E.T
    partials = []
    for chunk in range(NCHUNK):
        h_msg = _edge_messages(h_E_T, W_msg, b_msg, chunk)
        partials.append(_segment_sum(h_msg, src_idx, chunk))
    return _node_update(h_V, partials,
                        W_d, b_d, W_out, b_out, g1, be1, g2, be2)


# speedup vs baseline: 1.0559x; 1.0559x over previous
import functools

import jax
import jax.numpy as jnp
from jax import lax
from jax.experimental import pallas as pl
from jax.experimental.pallas import tpu as pltpu
from jax.experimental.pallas import tpu_sc as plsc

N = 10000
E = 320000
H = 128
NIN = 16
SCALE = 30.0
EPS = 1e-5

NUM_SC = 2
NUM_TILES = 16
NCHUNK = 2
ECHUNK = E // NCHUNK
EDGES_PER_TILE = ECHUNK // (NUM_SC * NUM_TILES)
CHUNK = 120
NRING = 3
NFULL = EDGES_PER_TILE // CHUNK
TAIL = EDGES_PER_TILE - NFULL * CHUNK
N_PAD = 10240
ROWS_PER_TILE = N_PAD // NUM_TILES


_EBLK = 6400

_SQRT_HALF = 0.7071067811865476


def _gelu(x):
    return 0.5 * x * (1.0 + lax.erf(x * _SQRT_HALF))


def _msg_body(het_ref, w_ref, b_ref, out_ref):
    m = lax.dot_general(het_ref[...].astype(jnp.bfloat16),
                        w_ref[...].astype(jnp.bfloat16),
                        ((( 0,), (0,)), ((), ())),
                        preferred_element_type=jnp.float32)
    out_ref[...] = _gelu(m + b_ref[...])


def _edge_messages(h_E_T, W_msg, b_msg, chunk):
    blk0 = chunk * (ECHUNK // _EBLK)
    return pl.pallas_call(
        _msg_body,
        grid=(ECHUNK // _EBLK,),
        in_specs=[
            pl.BlockSpec((H + NIN, _EBLK), lambda i: (0, i + blk0)),
            pl.BlockSpec((H + NIN, H), lambda i: (0, 0)),
            pl.BlockSpec((1, H), lambda i: (0, 0)),
        ],
        out_specs=pl.BlockSpec((_EBLK, H), lambda i: (i, 0)),
        out_shape=jax.ShapeDtypeStruct((ECHUNK, H), jnp.float32),
    )(h_E_T, W_msg, b_msg.reshape(1, H))


def _scatter_body(msg_hbm, idx_hbm, out_hbm, rows_v, idx_v, tidx_v,
                  acc_sh, lr, li, *, eoff):
    cid = lax.axis_index("c")
    sid = lax.axis_index("s")

    def _zrow(i, carry):
        for j in range(H // 16):
            rows_v[0, i, pl.ds(j * 16, 16)] = jnp.zeros((16,), jnp.float32)
        return carry

    lax.fori_loop(0, CHUNK, _zrow, 0)
    zbase = sid * ROWS_PER_TILE

    def _zcopy(k, carry):
        pltpu.sync_copy(rows_v.at[0], acc_sh.at[pl.ds(zbase + k * CHUNK, CHUNK)])
        return carry

    nz = ROWS_PER_TILE // CHUNK
    lax.fori_loop(0, nz, _zcopy, 0)
    rem = ROWS_PER_TILE - nz * CHUNK
    if rem:
        pltpu.sync_copy(rows_v.at[0, pl.ds(0, rem)],
                        acc_sh.at[pl.ds(zbase + nz * CHUNK, rem)])
    plsc.subcore_barrier()

    base = cid * (NUM_TILES * EDGES_PER_TILE) + sid * EDGES_PER_TILE

    def _start_load(c, b):
        off = base + c * CHUNK
        pltpu.async_copy(idx_hbm.at[pl.ds(eoff + off, CHUNK)], idx_v.at[b], li[b])
        pltpu.async_copy(msg_hbm.at[pl.ds(off, CHUNK)], rows_v.at[b], lr[b])

    def _wait_load(c, b):
        off = base + c * CHUNK
        pltpu.make_async_copy(idx_hbm.at[pl.ds(eoff + off, CHUNK)], idx_v.at[b], li[b]).wait()
        pltpu.make_async_copy(msg_hbm.at[pl.ds(off, CHUNK)], rows_v.at[b], lr[b]).wait()

    def _scatter(b):
        pltpu.sync_copy(rows_v.at[b], acc_sh.at[idx_v.at[b]], add=True)

    _start_load(0, 0)
    _start_load(1, 1)

    def _steady(c, carry, b):
        _wait_load(c, b)
        _start_load(c + 2, (b + 2) % 3)
        _scatter(b)
        return carry

    def _ring(k, carry):
        c0 = 3 * k
        for d in range(3):
            carry = _steady(c0 + d, carry, d)
        return carry

    nring = (NFULL - 2) // 3
    lax.fori_loop(0, nring, _ring, 0)
    for c in range(3 * nring, NFULL - 2):
        _steady(c, 0, c % 3)
    for c in (NFULL - 2, NFULL - 1):
        b = c % 3
        _wait_load(c, b)
        _scatter(b)
    if TAIL:
        off = base + NFULL * CHUNK
        pltpu.sync_copy(idx_hbm.at[pl.ds(eoff + off, TAIL)], tidx_v)
        pltpu.sync_copy(msg_hbm.at[pl.ds(off, TAIL)], rows_v.at[2, pl.ds(0, TAIL)])
        pltpu.sync_copy(rows_v.at[2, pl.ds(0, TAIL)], acc_sh.at[tidx_v], add=True)

    plsc.subcore_barrier()
    pltpu.sync_copy(acc_sh.at[pl.ds(sid * ROWS_PER_TILE, ROWS_PER_TILE)],
                    out_hbm.at[cid, pl.ds(sid * ROWS_PER_TILE, ROWS_PER_TILE)])


def _segment_sum(h_msg, src_idx, chunk):
    mesh = plsc.VectorSubcoreMesh(core_axis_name="c", subcore_axis_name="s")
    kern = functools.partial(
        pl.kernel,
        out_type=jax.ShapeDtypeStruct((NUM_SC, N_PAD, H), jnp.float32),
        mesh=mesh,
        scratch_types=[
            pltpu.VMEM((NRING, CHUNK, H), jnp.float32),
            pltpu.VMEM((NRING, CHUNK), jnp.int32),
            pltpu.VMEM((TAIL,), jnp.int32),
            pltpu.VMEM_SHARED((N_PAD, H), jnp.float32),
            [pltpu.SemaphoreType.DMA] * NRING,
            [pltpu.SemaphoreType.DMA] * NRING,
        ],
    )(functools.partial(_scatter_body, eoff=chunk * ECHUNK))
    return kern(h_msg, src_idx)


_NBLK = 2000


def _ln(x, g, b):
    mu = jnp.mean(x, axis=-1, keepdims=True)
    var = jnp.mean((x - mu) ** 2, axis=-1, keepdims=True)
    return (x - mu) * lax.rsqrt(var + EPS) * g + b


def _node_body(hv_ref, *refs):
    p_refs = refs[:NCHUNK]
    (wd_ref, bd_ref, wo_ref, bo_ref,
     g1_ref, be1_ref, g2_ref, be2_ref, out_ref) = refs[NCHUNK:]
    dh = p_refs[0][0] + p_refs[0][1]
    for p in p_refs[1:]:
        dh = dh + (p[0] + p[1])
    dh = dh * (1.0 / SCALE)
    h1 = _ln(hv_ref[...] + dh, g1_ref[...], be1_ref[...])
    d = jnp.dot(h1, wd_ref[...], preferred_element_type=jnp.float32) + bd_ref[...]
    d = _gelu(d)
    d = jnp.dot(d, wo_ref[...], preferred_element_type=jnp.float32) + bo_ref[...]
    out_ref[...] = _ln(h1 + d, g2_ref[...], be2_ref[...])


def _node_update(h_V, partials, W_d, b_d, W_out, b_out, g1, be1, g2, be2):
    row = lambda v: v.reshape(1, H)
    return pl.pallas_call(
        _node_body,
        grid=(N // _NBLK,),
        in_specs=[
            pl.BlockSpec((_NBLK, H), lambda i: (i, 0)),
        ] + [
            pl.BlockSpec((NUM_SC, _NBLK, H), lambda i: (0, i, 0))
            for _ in range(NCHUNK)
        ] + [
            pl.BlockSpec((H, H), lambda i: (0, 0)),
            pl.BlockSpec((1, H), lambda i: (0, 0)),
            pl.BlockSpec((H, H), lambda i: (0, 0)),
            pl.BlockSpec((1, H), lambda i: (0, 0)),
            pl.BlockSpec((1, H), lambda i: (0, 0)),
            pl.BlockSpec((1, H), lambda i: (0, 0)),
            pl.BlockSpec((1, H), lambda i: (0, 0)),
            pl.BlockSpec((1, H), lambda i: (0, 0)),
        ],
        out_specs=pl.BlockSpec((_NBLK, H), lambda i: (i, 0)),
        out_shape=jax.ShapeDtypeStruct((N, H), jnp.float32),
    )(h_V, *partials, W_d, row(b_d), W_out, row(b_out),
      row(g1), row(be1), row(g2), row(be2))


def kernel(h_V, h_E, edge_idx, W_msg, b_msg, W_d, b_d, W_out, b_out,
           g1, be1, g2, be2):
    src_idx = edge_idx[0].astype(jnp.int32)
    h_E_T = h_E.T
    partials = []
    for chunk in range(NCHUNK):
        h_msg = _edge_messages(h_E_T, W_msg, b_msg, chunk)
        partials.append(_segment_sum(h_msg, src_idx, chunk))
    return _node_update(h_V, partials,
                        W_d, b_d, W_out, b_out, g1, be1, g2, be2)
